# Initial kernel scaffold; baseline (speedup 1.0000x reference)
import jax
import jax.numpy as jnp
from jax.experimental import pallas as pl


def kernel(z1, z2):
    raise NotImplementedError("write your pallas kernel here")


# speedup vs baseline: 68.1165x; 68.1165x over previous
import numpy as np
import jax
import jax.numpy as jnp
from jax.experimental import pallas as pl
from jax.experimental.pallas import tpu as pltpu

_LOG_SQRT_2PI = np.float32(0.5 * np.log(2.0 * np.pi))


def _diag_kernel_r2(z1_ref, z2_ref, out_ref):
    K, B, M = z2_ref.shape
    z1b = z1_ref[...]
    means = z1b[:, :M]
    logvar = z1b[:, M:]
    inv_std = jnp.exp(-0.5 * logvar)
    base = -0.5 * logvar - _LOG_SQRT_2PI
    acc = jnp.float32(0.0)
    for k in range(K):
        d = (z2_ref[k] - means) * inv_std
        acc = acc + jnp.sum(jnp.exp(base - 0.5 * d * d))
    out_ref[...] = (acc / np.float32(B * K * M)).reshape(1, 1)


def kernel(z1, z2):
    B = z1.shape[0]
    M = z2.shape[1]
    K = z2.shape[0] // B
    z2r = z2.reshape(K, B, M)
    out = pl.pallas_call(
        _diag_kernel_r2,
        out_shape=jax.ShapeDtypeStruct((1, 1), jnp.float32),
    )(z1, z2r)
    return out.reshape(())
